# Initial kernel scaffold; baseline (speedup 1.0000x reference)
import jax
import jax.numpy as jnp
from jax.experimental import pallas as pl


def kernel(in_feat, edge_index, W_self1, W_neigh1, b1, W_self2, W_neigh2, b2, W_self3, W_neigh3, b3, W_self4, W_neigh4, b4):
    raise NotImplementedError("write your pallas kernel here")


# speedup vs baseline: 4.7716x; 4.7716x over previous
import jax
import jax.numpy as jnp
from jax import lax
from jax.experimental import pallas as pl
from jax.experimental.pallas import tpu as pltpu
from jax.experimental.pallas import tpu_sc as plsc

NC = 2
NS = 16
CHUNK = 80
D = 128
F32 = jnp.float32


def _sc_agg(n, e, gather):
  epw = e // (NC * NS)
  nchunk = epw // CHUNK
  rpt = n // NS
  assert epw % CHUNK == 0 and n % NS == 0 and CHUNK % 8 == 0

  mesh = plsc.VectorSubcoreMesh(core_axis_name="c", subcore_axis_name="s")
  out_type = jax.ShapeDtypeStruct((NC, n, D), F32)
  scratch = [
      pltpu.VMEM((CHUNK,), jnp.int32),
      pltpu.VMEM((CHUNK, D), F32),
      pltpu.VMEM_SHARED((n, D), F32),
  ]
  if gather:
    scratch.append(pltpu.VMEM((CHUNK,), jnp.int32))

  def body(*refs):
    if gather:
      (table, gidx_hbm, sidx_hbm, zrows_hbm, p_out,
       sidx, rows, acc, gidx) = refs
    else:
      (ones_hbm, sidx_hbm, zrows_hbm, p_out, sidx, rows, acc) = refs

    c = lax.axis_index("c")
    s = lax.axis_index("s")
    base = (c * NS + s) * epw
    row0 = s * rpt

    pltpu.sync_copy(zrows_hbm, acc.at[pl.ds(row0, rpt)])
    if not gather:
      pltpu.sync_copy(ones_hbm, rows)
    plsc.subcore_barrier()

    @pl.loop(0, nchunk)
    def _(k):
      off = pl.multiple_of(base + k * CHUNK, 8)
      pltpu.sync_copy(sidx_hbm.at[pl.ds(off, CHUNK)], sidx)
      if gather:
        pltpu.sync_copy(gidx_hbm.at[pl.ds(off, CHUNK)], gidx)
        pltpu.sync_copy(table.at[gidx], rows)
      pltpu.sync_copy(rows, acc.at[sidx], add=True)

    plsc.subcore_barrier()
    rb = rpt // 8 * 8
    tail = n - rb * NS
    wb0 = s * rb
    pltpu.sync_copy(acc.at[pl.ds(wb0, rb)], p_out.at[c, pl.ds(wb0, rb)])
    if tail:
      @pl.when(s == 0)
      def _():
        pltpu.sync_copy(acc.at[pl.ds(rb * NS, tail)],
                        p_out.at[c, pl.ds(rb * NS, tail)])

  return pl.kernel(body, out_type=out_type, mesh=mesh, scratch_types=scratch)


def _dot(a, b):
  return jnp.dot(a, b, preferred_element_type=F32)


def _tc_first(x, ws, wn, b):
  n = x.shape[0]

  def body(x_ref, ws_ref, wn_ref, b_ref, s_out, z_out):
    xv = x_ref[...]
    s_out[...] = _dot(xv, ws_ref[...]) + b_ref[...]
    z_out[...] = _dot(xv, wn_ref[...])

  return pl.pallas_call(
      body,
      out_shape=[jax.ShapeDtypeStruct((n, D), F32),
                 jax.ShapeDtypeStruct((n, D), F32)],
  )(x, ws, wn, b)


def _tc_mid(sprev, p, dg_or_inv, ws, wn, b, emit_inv):
  n = sprev.shape[0]
  out_shape = [jax.ShapeDtypeStruct((n, D), F32),
               jax.ShapeDtypeStruct((n, D), F32)]
  if emit_inv:
    out_shape.append(jax.ShapeDtypeStruct((n, D), F32))

  def body(s_ref, p_ref, d_ref, ws_ref, wn_ref, b_ref, *outs):
    if emit_inv:
      s_out, z_out, inv_out = outs
      inv = 1.0 / jnp.maximum(d_ref[0] + d_ref[1], 1.0)
      inv_out[...] = inv
    else:
      s_out, z_out = outs
      inv = d_ref[...]
    h = jnp.maximum(s_ref[...] + inv * (p_ref[0] + p_ref[1]), 0.0)
    s_out[...] = _dot(h, ws_ref[...]) + b_ref[...]
    z_out[...] = _dot(h, wn_ref[...])

  return pl.pallas_call(body, out_shape=out_shape)(
      sprev, p, dg_or_inv, ws, wn, b)


def _tc_final(sprev, p, invr, cpart, ws, wn, b):
  n = sprev.shape[0]
  dout = ws.shape[1]

  def body(s_ref, p_ref, inv_ref, c_ref, ws_ref, wn_ref, b_ref, out_ref):
    h = jnp.maximum(s_ref[...] + inv_ref[...] * (p_ref[0] + p_ref[1]), 0.0)
    cc = c_ref[0] + c_ref[1]
    u = jnp.sum(h, axis=0, keepdims=True) * (1.0 / n)
    w = jnp.sum(h * cc, axis=0, keepdims=True) * (1.0 / n)
    out_ref[...] = _dot(u, ws_ref[...]) + _dot(w, wn_ref[...]) + b_ref[...]

  return pl.pallas_call(
      body, out_shape=jax.ShapeDtypeStruct((1, dout), F32),
  )(sprev, p, invr, cpart, ws, wn, b)


def _pad(w):
  return jnp.pad(w, ((0, D - w.shape[0]), (0, D - w.shape[1])))


def kernel(in_feat, edge_index, W_self1, W_neigh1, b1, W_self2, W_neigh2, b2,
           W_self3, W_neigh3, b3, W_self4, W_neigh4, b4):
  n, _ = in_feat.shape
  e = edge_index.shape[1]
  src = edge_index[0]
  dst = edge_index[1]
  rpt = n // NS

  ws2, wn2 = _pad(W_self2), _pad(W_neigh2)
  ws3, wn3 = _pad(W_self3), _pad(W_neigh3)
  ws4 = jnp.pad(W_self4, ((0, D - W_self4.shape[0]), (0, 0)))
  wn4 = jnp.pad(W_neigh4, ((0, D - W_neigh4.shape[0]), (0, 0)))
  b1p = b1.reshape(1, -1)
  b2p = jnp.pad(b2, (0, D - b2.shape[0])).reshape(1, -1)
  b3p = jnp.pad(b3, (0, D - b3.shape[0])).reshape(1, -1)
  b4p = b4.reshape(1, -1)

  zrows = jnp.zeros((rpt, D), F32)
  ones_rows = jnp.ones((CHUNK, D), F32)

  agg = _sc_agg(n, e, gather=True)
  ones_agg = _sc_agg(n, e, gather=False)

  s1, z1 = _tc_first(in_feat, W_self1, W_neigh1, b1p)
  p1 = agg(z1, src, dst, zrows)
  deg = ones_agg(ones_rows, dst, zrows)
  s2, z2, invr = _tc_mid(s1, p1, deg, ws2, wn2, b2p, True)
  p2 = agg(z2, src, dst, zrows)
  cpart = agg(invr, dst, src, zrows)
  s3, z3 = _tc_mid(s2, p2, invr, ws3, wn3, b3p, False)
  p3 = agg(z3, src, dst, zrows)
  return _tc_final(s3, p3, invr, cpart, ws4, wn4, b4p)


# speedup vs baseline: 12.5185x; 2.6236x over previous
import jax
import jax.numpy as jnp
from jax import lax
from jax.experimental import pallas as pl
from jax.experimental.pallas import tpu as pltpu
from jax.experimental.pallas import tpu_sc as plsc

NC = 2
NS = 16
CHUNK = 125
WIN = 8
D = 128
NP = 10240
F32 = jnp.float32


def _sc_agg(e, mode):
  epw = e // (NC * NS)
  nchunk = epw // CHUNK
  nwin = nchunk // WIN
  rpt = NP // NS
  npw = NP // NS
  assert epw % CHUNK == 0 and nchunk % WIN == 0

  mesh = plsc.VectorSubcoreMesh(core_axis_name="c", subcore_axis_name="s")
  out_type = [jax.ShapeDtypeStruct((NC, NP, D), F32)]
  if mode != "plain":
    out_type.append(jax.ShapeDtypeStruct((NC, 1, NP), F32))

  scratch = [
      pltpu.VMEM((WIN, CHUNK), jnp.int32),
      pltpu.VMEM((WIN, CHUNK), jnp.int32),
      pltpu.VMEM((CHUNK, D), F32),
      pltpu.VMEM((CHUNK, D), F32),
      pltpu.VMEM_SHARED((NP, D), F32),
      pltpu.SemaphoreType.DMA,
      pltpu.SemaphoreType.DMA,
  ]
  if mode == "deg":
    scratch += [pltpu.VMEM((CHUNK,), F32),
                pltpu.VMEM_SHARED((NP,), F32)]
  elif mode == "c":
    scratch += [pltpu.VMEM((CHUNK,), F32),
                pltpu.VMEM_SHARED((NP,), F32)]

  def body(*refs):
    if mode == "plain":
      (table, gidx_hbm, sidx_hbm, zrows_hbm, z1d_hbm, p_out,
       sidxw, gidxw, rows0, rows1, acc, gsem0, gsem1) = refs
    elif mode == "deg":
      (table, gidx_hbm, sidx_hbm, zrows_hbm, z1d_hbm, ones_hbm,
       p_out, aux_out,
       sidxw, gidxw, rows0, rows1, acc, gsem0, gsem1, vals, aux) = refs
    else:
      (table, inv_hbm, gidx_hbm, sidx_hbm, zrows_hbm, z1d_hbm,
       p_out, aux_out,
       sidxw, gidxw, rows0, rows1, acc, gsem0, gsem1, vals, aux) = refs

    c = lax.axis_index("c")
    s = lax.axis_index("s")
    wid = c * NS + s
    row0 = s * rpt

    pltpu.sync_copy(zrows_hbm, acc.at[pl.ds(row0, rpt)])
    if mode != "plain":
      pltpu.sync_copy(z1d_hbm, aux.at[pl.ds(s * npw, npw)])
    if mode == "deg":
      pltpu.sync_copy(ones_hbm, vals)
    plsc.subcore_barrier()

    bufs = [(rows0, gsem0), (rows1, gsem1)]

    @pl.loop(0, nwin)
    def _(w):
      w0 = pl.multiple_of(w * WIN, WIN)
      pltpu.sync_copy(sidx_hbm.at[wid, pl.ds(w0, WIN)], sidxw)
      pltpu.sync_copy(gidx_hbm.at[wid, pl.ds(w0, WIN)], gidxw)
      pltpu.async_copy(table.at[gidxw.at[0]], rows0, gsem0)
      for j in range(WIN):
        buf, sem = bufs[j % 2]
        pltpu.make_async_copy(table.at[gidxw.at[j]], buf, sem).wait()
        if j + 1 < WIN:
          nbuf, nsem = bufs[(j + 1) % 2]
          pltpu.async_copy(table.at[gidxw.at[j + 1]], nbuf, nsem)
        pltpu.sync_copy(buf, acc.at[sidxw.at[j]], add=True)
        if mode == "deg":
          pltpu.sync_copy(vals, aux.at[sidxw.at[j]], add=True)
        elif mode == "c":
          pltpu.sync_copy(inv_hbm.at[sidxw.at[j]], vals)
          pltpu.sync_copy(vals, aux.at[gidxw.at[j]], add=True)

    plsc.subcore_barrier()
    pltpu.sync_copy(acc.at[pl.ds(row0, rpt)], p_out.at[c, pl.ds(row0, rpt)])
    if mode != "plain":
      pltpu.sync_copy(aux.at[pl.ds(s * npw, npw)],
                      aux_out.at[c, 0, pl.ds(s * npw, npw)])

  return pl.kernel(body, out_type=out_type, mesh=mesh, scratch_types=scratch)


def _dot(a, b):
  return jnp.dot(a, b, preferred_element_type=F32)


def _tc_first(x, ws, wn, b):
  def body(x_ref, ws_ref, wn_ref, b_ref, s_out, z_out):
    xv = x_ref[...]
    s_out[...] = _dot(xv, ws_ref[...]) + b_ref[...]
    z_out[...] = _dot(xv, wn_ref[...])

  return pl.pallas_call(
      body,
      out_shape=[jax.ShapeDtypeStruct((NP, D), F32),
                 jax.ShapeDtypeStruct((NP, D), F32)],
  )(x, ws, wn, b)


def _tc_mid(sprev, p, dg_or_inv, ws, wn, b, emit_inv):
  out_shape = [jax.ShapeDtypeStruct((NP, D), F32),
               jax.ShapeDtypeStruct((NP, D), F32)]
  if emit_inv:
    out_shape += [jax.ShapeDtypeStruct((NP, D), F32),
                  jax.ShapeDtypeStruct((NP,), F32)]

  def body(s_ref, p_ref, d_ref, ws_ref, wn_ref, b_ref, *outs):
    if emit_inv:
      s_out, z_out, invr_out, inv1_out = outs
      invm = 1.0 / jnp.maximum(d_ref[0, 0, :] + d_ref[1, 0, :], 1.0)
      inv1_out[...] = invm
      inv = lax.broadcast_in_dim(invm, (NP, D), (0,))
      invr_out[...] = inv
    else:
      s_out, z_out = outs
      inv = d_ref[...]
    h = jnp.maximum(s_ref[...] + inv * (p_ref[0] + p_ref[1]), 0.0)
    s_out[...] = _dot(h, ws_ref[...]) + b_ref[...]
    z_out[...] = _dot(h, wn_ref[...])

  return pl.pallas_call(body, out_shape=out_shape)(
      sprev, p, dg_or_inv, ws, wn, b)


def _tc_final(sprev, p, invr, cpart, ws, wn, b, n):
  dout = ws.shape[1]

  def body(s_ref, p_ref, inv_ref, c_ref, ws_ref, wn_ref, b_ref, out_ref):
    h = jnp.maximum(s_ref[...] + inv_ref[...] * (p_ref[0] + p_ref[1]), 0.0)
    cs = (c_ref[0, 0, :] + c_ref[1, 0, :]).reshape(1, NP)
    u = jnp.sum(h, axis=0, keepdims=True) * (1.0 / n)
    w = _dot(cs, h) * (1.0 / n)
    out_ref[...] = _dot(u, ws_ref[...]) + _dot(w, wn_ref[...]) + b_ref[...]

  return pl.pallas_call(
      body, out_shape=jax.ShapeDtypeStruct((1, dout), F32),
  )(sprev, p, invr, cpart, ws, wn, b)


def _pad(w):
  return jnp.pad(w, ((0, D - w.shape[0]), (0, D - w.shape[1])))


def kernel(in_feat, edge_index, W_self1, W_neigh1, b1, W_self2, W_neigh2, b2,
           W_self3, W_neigh3, b3, W_self4, W_neigh4, b4):
  n, _ = in_feat.shape
  e = edge_index.shape[1]
  nchunk = e // (NC * NS) // CHUNK
  src = edge_index[0].reshape(NC * NS, nchunk, CHUNK)
  dst = edge_index[1].reshape(NC * NS, nchunk, CHUNK)

  xp = jnp.pad(in_feat, ((0, NP - n), (0, 0)))
  ws2, wn2 = _pad(W_self2), _pad(W_neigh2)
  ws3, wn3 = _pad(W_self3), _pad(W_neigh3)
  ws4 = jnp.pad(W_self4, ((0, D - W_self4.shape[0]), (0, 0)))
  wn4 = jnp.pad(W_neigh4, ((0, D - W_neigh4.shape[0]), (0, 0)))
  b1p = b1.reshape(1, -1)
  b2p = jnp.pad(b2, (0, D - b2.shape[0])).reshape(1, -1)
  b3p = jnp.pad(b3, (0, D - b3.shape[0])).reshape(1, -1)
  b4p = b4.reshape(1, -1)

  zrows = jnp.zeros((NP // NS, D), F32)
  z1d = jnp.zeros((NP // NS,), F32)
  ones1 = jnp.ones((CHUNK,), F32)

  s1, z1 = _tc_first(xp, W_self1, W_neigh1, b1p)
  p1, deg = _sc_agg(e, "deg")(z1, src, dst, zrows, z1d, ones1)
  s2, z2, invr, inv1 = _tc_mid(s1, p1, deg, ws2, wn2, b2p, True)
  p2, cpart = _sc_agg(e, "c")(z2, inv1, src, dst, zrows, z1d)
  s3, z3 = _tc_mid(s2, p2, invr, ws3, wn3, b3p, False)
  p3 = _sc_agg(e, "plain")(z3, src, dst, zrows, z1d)[0]
  return _tc_final(s3, p3, invr, cpart, ws4, wn4, b4p, n)


# speedup vs baseline: 15.2019x; 1.2144x over previous
import jax
import jax.numpy as jnp
from jax import lax
from jax.experimental import pallas as pl
from jax.experimental.pallas import tpu as pltpu
from jax.experimental.pallas import tpu_sc as plsc

NC = 2
NS = 16
CHUNK = 125
WIN = 8
D = 128
NP = 10240
F32 = jnp.float32


def _sc_agg(e, mode):
  epw = e // (NC * NS)
  nchunk = epw // CHUNK
  nwin = nchunk // WIN
  rpt = NP // NS
  npw = NP // NS
  assert epw % CHUNK == 0 and nchunk % WIN == 0 and (nchunk // WIN) % 2 == 0

  mesh = plsc.VectorSubcoreMesh(core_axis_name="c", subcore_axis_name="s")
  out_type = [jax.ShapeDtypeStruct((NC, NP, D), F32)]
  if mode != "plain":
    out_type.append(jax.ShapeDtypeStruct((NC, 1, NP), F32))

  scratch = [
      pltpu.VMEM((WIN, CHUNK), jnp.int32),
      pltpu.VMEM((WIN, CHUNK), jnp.int32),
      pltpu.VMEM((WIN, CHUNK), jnp.int32),
      pltpu.VMEM((WIN, CHUNK), jnp.int32),
      pltpu.VMEM((CHUNK, D), F32),
      pltpu.VMEM((CHUNK, D), F32),
      pltpu.VMEM_SHARED((NP, D), F32),
      pltpu.SemaphoreType.DMA,
      pltpu.SemaphoreType.DMA,
      pltpu.SemaphoreType.DMA,
      pltpu.SemaphoreType.DMA,
      pltpu.SemaphoreType.DMA,
      pltpu.SemaphoreType.DMA,
  ]
  if mode == "deg":
    scratch += [pltpu.VMEM((CHUNK,), F32),
                pltpu.VMEM_SHARED((NP,), F32)]
  elif mode == "c":
    scratch += [pltpu.VMEM((CHUNK,), F32),
                pltpu.VMEM_SHARED((NP,), F32),
                pltpu.VMEM_SHARED((NP,), F32)]

  def body(*refs):
    if mode == "plain":
      (table, gidx_hbm, sidx_hbm, zrows_hbm, z1d_hbm, p_out,
       *scr) = refs
    elif mode == "deg":
      (table, gidx_hbm, sidx_hbm, zrows_hbm, z1d_hbm, ones_hbm,
       p_out, aux_out, *scr) = refs
    else:
      (table, inv_hbm, gidx_hbm, sidx_hbm, zrows_hbm, z1d_hbm,
       p_out, aux_out, *scr) = refs
    (sidxw0, gidxw0, sidxw1, gidxw1, rows0, rows1, acc,
     gsem0, gsem1, isem0, isem1, ssem0, ssem1) = scr[:13]
    if mode == "deg":
      vals, aux = scr[13:]
    elif mode == "c":
      vals, aux, invt = scr[13:]

    c = lax.axis_index("c")
    s = lax.axis_index("s")
    wid = c * NS + s
    row0 = s * rpt

    wbufs = [(sidxw0, gidxw0, isem0), (sidxw1, gidxw1, isem1)]

    def idx_start(w, wb):
      sw, gw, isem = wb
      w0 = pl.multiple_of(w * WIN, WIN)
      pltpu.async_copy(sidx_hbm.at[wid, pl.ds(w0, WIN)], sw, isem)
      pltpu.async_copy(gidx_hbm.at[wid, pl.ds(w0, WIN)], gw, isem)

    def idx_wait(w, wb):
      sw, gw, isem = wb
      w0 = pl.multiple_of(w * WIN, WIN)
      pltpu.make_async_copy(sidx_hbm.at[wid, pl.ds(w0, WIN)], sw, isem).wait()
      pltpu.make_async_copy(gidx_hbm.at[wid, pl.ds(w0, WIN)], gw, isem).wait()

    idx_start(0, wbufs[0])
    pltpu.sync_copy(zrows_hbm, acc.at[pl.ds(row0, rpt)])
    if mode != "plain":
      pltpu.sync_copy(z1d_hbm, aux.at[pl.ds(s * npw, npw)])
    if mode == "deg":
      pltpu.sync_copy(ones_hbm, vals)
    elif mode == "c":
      pltpu.sync_copy(inv_hbm.at[pl.ds(s * npw, npw)],
                      invt.at[pl.ds(s * npw, npw)])
    plsc.subcore_barrier()

    bufs = [(rows0, gsem0, ssem0), (rows1, gsem1, ssem1)]

    @pl.loop(0, nwin // 2)
    def _(ww):
      for par in range(2):
        w = ww * 2 + par
        sidxw, gidxw, _ = wbufs[par]
        idx_wait(w, wbufs[par])

        @pl.when(w + 1 < nwin)
        def _():
          idx_start(w + 1, wbufs[1 - par])

        pltpu.async_copy(table.at[gidxw.at[0]], rows0, gsem0)
        for j in range(WIN):
          buf, gsem, ssem = bufs[j % 2]
          if j + 1 < WIN:
            nbuf, ngsem, nssem = bufs[(j + 1) % 2]
            if j >= 1:
              pltpu.make_async_copy(nbuf, acc.at[pl.ds(0, CHUNK)],
                                    nssem).wait()
            pltpu.async_copy(table.at[gidxw.at[j + 1]], nbuf, ngsem)
          pltpu.make_async_copy(table.at[gidxw.at[j]], buf, gsem).wait()
          pltpu.async_copy(buf, acc.at[sidxw.at[j]], ssem, add=True)
          if mode == "deg":
            pltpu.sync_copy(vals, aux.at[sidxw.at[j]], add=True)
          elif mode == "c":
            pltpu.sync_copy(invt.at[sidxw.at[j]], vals)
            pltpu.sync_copy(vals, aux.at[gidxw.at[j]], add=True)
        pltpu.make_async_copy(rows0, acc.at[pl.ds(0, CHUNK)], ssem0).wait()
        pltpu.make_async_copy(rows1, acc.at[pl.ds(0, CHUNK)], ssem1).wait()

    plsc.subcore_barrier()
    pltpu.sync_copy(acc.at[pl.ds(row0, rpt)], p_out.at[c, pl.ds(row0, rpt)])
    if mode != "plain":
      pltpu.sync_copy(aux.at[pl.ds(s * npw, npw)],
                      aux_out.at[c, 0, pl.ds(s * npw, npw)])

  return pl.kernel(body, out_type=out_type, mesh=mesh, scratch_types=scratch)


def _dot(a, b):
  return jnp.dot(a, b, preferred_element_type=F32)


def _inv_bcast(d_ref):
  invm = 1.0 / jnp.maximum(d_ref[0, 0, :] + d_ref[1, 0, :], 1.0)
  return invm, lax.broadcast_in_dim(invm, (NP, D), (0,))


BR = 2048


def _tc_matmul(x, w, b):
  grid = (NP // BR,)
  bspec = pl.BlockSpec((BR, D), lambda i: (i, 0))
  wspec = pl.BlockSpec((D, D), lambda i: (0, 0))
  if b is None:
    def body(x_ref, w_ref, z_out):
      z_out[...] = _dot(x_ref[...], w_ref[...])
    args, specs = (x, w), [bspec, wspec]
  else:
    def body(x_ref, w_ref, b_ref, z_out):
      z_out[...] = _dot(x_ref[...], w_ref[...]) + b_ref[...]
    args, specs = (x, w, b), [bspec, wspec, pl.BlockSpec((1, D), lambda i: (0, 0))]
  return pl.pallas_call(
      body, grid=grid, in_specs=specs, out_specs=bspec,
      out_shape=jax.ShapeDtypeStruct((NP, D), F32))(*args)


def _tc_comb(sprev, p, deg, wn, emit_inv1):
  grid = (NP // BR,)
  bspec = pl.BlockSpec((BR, D), lambda i: (i, 0))
  out_shape = [jax.ShapeDtypeStruct((NP, D), F32),
               jax.ShapeDtypeStruct((NP, D), F32)]
  out_specs = [bspec, bspec]
  if emit_inv1:
    out_shape.append(jax.ShapeDtypeStruct((NP,), F32))
    out_specs.append(pl.BlockSpec((BR,), lambda i: (i,)))

  def body(s_ref, p_ref, d_ref, wn_ref, *outs):
    invm = 1.0 / jnp.maximum(d_ref[0, 0, :] + d_ref[1, 0, :], 1.0)
    inv = lax.broadcast_in_dim(invm, (BR, D), (0,))
    if emit_inv1:
      h_out, z_out, inv1_out = outs
      inv1_out[...] = invm
    else:
      h_out, z_out = outs
    h = jnp.maximum(s_ref[...] + inv * (p_ref[0] + p_ref[1]), 0.0)
    h_out[...] = h
    z_out[...] = _dot(h, wn_ref[...])

  return pl.pallas_call(
      body, grid=grid,
      in_specs=[bspec, pl.BlockSpec((NC, BR, D), lambda i: (0, i, 0)),
                pl.BlockSpec((NC, 1, BR), lambda i: (0, 0, i)),
                pl.BlockSpec((D, D), lambda i: (0, 0))],
      out_specs=out_specs, out_shape=out_shape)(sprev, p, deg, wn)


def _tc_final(sprev, p, deg, cpart, ws, wn, b, n):
  dout = ws.shape[1]

  def body(s_ref, p_ref, d_ref, c_ref, ws_ref, wn_ref, b_ref, out_ref):
    _, inv = _inv_bcast(d_ref)
    h = jnp.maximum(s_ref[...] + inv * (p_ref[0] + p_ref[1]), 0.0)
    cs = (c_ref[0, 0, :] + c_ref[1, 0, :]).reshape(1, NP)
    u = jnp.sum(h, axis=0, keepdims=True) * (1.0 / n)
    w = _dot(cs, h) * (1.0 / n)
    out_ref[...] = _dot(u, ws_ref[...]) + _dot(w, wn_ref[...]) + b_ref[...]

  return pl.pallas_call(
      body, out_shape=jax.ShapeDtypeStruct((1, dout), F32),
  )(sprev, p, deg, cpart, ws, wn, b)


def _pad(w):
  return jnp.pad(w, ((0, D - w.shape[0]), (0, D - w.shape[1])))


def kernel(in_feat, edge_index, W_self1, W_neigh1, b1, W_self2, W_neigh2, b2,
           W_self3, W_neigh3, b3, W_self4, W_neigh4, b4):
  n, _ = in_feat.shape
  e = edge_index.shape[1]
  nchunk = e // (NC * NS) // CHUNK
  src = edge_index[0].reshape(NC * NS, nchunk, CHUNK)
  dst = edge_index[1].reshape(NC * NS, nchunk, CHUNK)

  xp = jnp.pad(in_feat, ((0, NP - n), (0, 0)))
  ws2, wn2 = _pad(W_self2), _pad(W_neigh2)
  ws3, wn3 = _pad(W_self3), _pad(W_neigh3)
  ws4 = jnp.pad(W_self4, ((0, D - W_self4.shape[0]), (0, 0)))
  wn4 = jnp.pad(W_neigh4, ((0, D - W_neigh4.shape[0]), (0, 0)))
  b1p = b1.reshape(1, -1)
  b2p = jnp.pad(b2, (0, D - b2.shape[0])).reshape(1, -1)
  b3p = jnp.pad(b3, (0, D - b3.shape[0])).reshape(1, -1)
  b4p = b4.reshape(1, -1)

  zrows = jnp.zeros((NP // NS, D), F32)
  z1d = jnp.zeros((NP // NS,), F32)
  ones1 = jnp.ones((CHUNK,), F32)

  z1 = _tc_matmul(xp, W_neigh1, None)
  s1 = _tc_matmul(xp, W_self1, b1p)
  p1, deg = _sc_agg(e, "deg")(z1, src, dst, zrows, z1d, ones1)
  h1, z2, inv1 = _tc_comb(s1, p1, deg, wn2, True)
  s2 = _tc_matmul(h1, ws2, b2p)
  p2, cpart = _sc_agg(e, "c")(z2, inv1, src, dst, zrows, z1d)
  h2, z3 = _tc_comb(s2, p2, deg, wn3, False)
  s3 = _tc_matmul(h2, ws3, b3p)
  p3 = _sc_agg(e, "plain")(z3, src, dst, zrows, z1d)[0]
  return _tc_final(s3, p3, deg, cpart, ws4, wn4, b4p, n)


# speedup vs baseline: 15.2782x; 1.0050x over previous
import jax
import jax.numpy as jnp
from jax import lax
from jax.experimental import pallas as pl
from jax.experimental.pallas import tpu as pltpu
from jax.experimental.pallas import tpu_sc as plsc

NC = 2
NS = 16
CHUNK = 125
WIN = 8
D = 128
NP = 10240
F32 = jnp.float32


def _sc_agg(e, mode):
  epw = e // (NC * NS)
  nchunk = epw // CHUNK
  nwin = nchunk // WIN
  rpt = NP // NS
  npw = NP // NS
  assert epw % CHUNK == 0 and nchunk % WIN == 0 and (nchunk // WIN) % 2 == 0

  mesh = plsc.VectorSubcoreMesh(core_axis_name="c", subcore_axis_name="s")
  out_type = [jax.ShapeDtypeStruct((NC, NP, D), F32)]
  if mode != "plain":
    out_type.append(jax.ShapeDtypeStruct((NC, 1, NP), F32))

  scratch = [
      pltpu.VMEM((WIN, CHUNK), jnp.int32),
      pltpu.VMEM((WIN, CHUNK), jnp.int32),
      pltpu.VMEM((WIN, CHUNK), jnp.int32),
      pltpu.VMEM((WIN, CHUNK), jnp.int32),
      pltpu.VMEM((CHUNK, D), F32),
      pltpu.VMEM((CHUNK, D), F32),
      pltpu.VMEM_SHARED((NP, D), F32),
      pltpu.SemaphoreType.DMA,
      pltpu.SemaphoreType.DMA,
      pltpu.SemaphoreType.DMA,
      pltpu.SemaphoreType.DMA,
      pltpu.SemaphoreType.DMA,
      pltpu.SemaphoreType.DMA,
  ]
  if mode == "deg":
    scratch += [pltpu.VMEM((CHUNK,), F32),
                pltpu.VMEM_SHARED((NP,), F32)]
  elif mode == "c":
    scratch += [pltpu.VMEM((CHUNK,), F32),
                pltpu.VMEM_SHARED((NP,), F32),
                pltpu.VMEM_SHARED((NP,), F32)]

  def body(*refs):
    if mode == "plain":
      (table, gidx_hbm, sidx_hbm, zrows_hbm, z1d_hbm, p_out,
       *scr) = refs
    elif mode == "deg":
      (table, gidx_hbm, sidx_hbm, zrows_hbm, z1d_hbm, ones_hbm,
       p_out, aux_out, *scr) = refs
    else:
      (table, inv_hbm, gidx_hbm, sidx_hbm, zrows_hbm, z1d_hbm,
       p_out, aux_out, *scr) = refs
    (sidxw0, gidxw0, sidxw1, gidxw1, rows0, rows1, acc,
     gsem0, gsem1, isem0, isem1, ssem0, ssem1) = scr[:13]
    if mode == "deg":
      vals, aux = scr[13:]
    elif mode == "c":
      vals, aux, invt = scr[13:]

    c = lax.axis_index("c")
    s = lax.axis_index("s")
    wid = c * NS + s
    row0 = s * rpt

    wbufs = [(sidxw0, gidxw0, isem0), (sidxw1, gidxw1, isem1)]

    def idx_start(w, wb):
      sw, gw, isem = wb
      w0 = pl.multiple_of(w * WIN, WIN)
      pltpu.async_copy(sidx_hbm.at[wid, pl.ds(w0, WIN)], sw, isem)
      pltpu.async_copy(gidx_hbm.at[wid, pl.ds(w0, WIN)], gw, isem)

    def idx_wait(w, wb):
      sw, gw, isem = wb
      w0 = pl.multiple_of(w * WIN, WIN)
      pltpu.make_async_copy(sidx_hbm.at[wid, pl.ds(w0, WIN)], sw, isem).wait()
      pltpu.make_async_copy(gidx_hbm.at[wid, pl.ds(w0, WIN)], gw, isem).wait()

    idx_start(0, wbufs[0])
    pltpu.sync_copy(zrows_hbm, acc.at[pl.ds(row0, rpt)])
    if mode != "plain":
      pltpu.sync_copy(z1d_hbm, aux.at[pl.ds(s * npw, npw)])
    if mode == "deg":
      pltpu.sync_copy(ones_hbm, vals)
    elif mode == "c":
      pltpu.sync_copy(inv_hbm.at[pl.ds(s * npw, npw)],
                      invt.at[pl.ds(s * npw, npw)])
    plsc.subcore_barrier()

    bufs = [(rows0, gsem0, ssem0), (rows1, gsem1, ssem1)]

    @pl.loop(0, nwin // 2)
    def _(ww):
      for par in range(2):
        w = ww * 2 + par
        sidxw, gidxw, _ = wbufs[par]
        idx_wait(w, wbufs[par])

        @pl.when(w + 1 < nwin)
        def _():
          idx_start(w + 1, wbufs[1 - par])

        pltpu.async_copy(table.at[gidxw.at[0]], rows0, gsem0)
        for j in range(WIN):
          buf, gsem, ssem = bufs[j % 2]
          if j + 1 < WIN:
            nbuf, ngsem, nssem = bufs[(j + 1) % 2]
            if j >= 1:
              pltpu.make_async_copy(nbuf, acc.at[pl.ds(0, CHUNK)],
                                    nssem).wait()
            pltpu.async_copy(table.at[gidxw.at[j + 1]], nbuf, ngsem)
          pltpu.make_async_copy(table.at[gidxw.at[j]], buf, gsem).wait()
          pltpu.async_copy(buf, acc.at[sidxw.at[j]], ssem, add=True)
          if mode == "deg":
            pltpu.sync_copy(vals, aux.at[sidxw.at[j]], add=True)
          elif mode == "c":
            pltpu.sync_copy(invt.at[sidxw.at[j]], vals)
            pltpu.sync_copy(vals, aux.at[gidxw.at[j]], add=True)
        pltpu.make_async_copy(rows0, acc.at[pl.ds(0, CHUNK)], ssem0).wait()
        pltpu.make_async_copy(rows1, acc.at[pl.ds(0, CHUNK)], ssem1).wait()

    plsc.subcore_barrier()
    pltpu.sync_copy(acc.at[pl.ds(row0, rpt)], p_out.at[c, pl.ds(row0, rpt)])
    if mode != "plain":
      pltpu.sync_copy(aux.at[pl.ds(s * npw, npw)],
                      aux_out.at[c, 0, pl.ds(s * npw, npw)])

  return pl.kernel(body, out_type=out_type, mesh=mesh, scratch_types=scratch)


def _dot(a, b):
  return jnp.dot(a, b, preferred_element_type=F32)


def _inv_bcast(d_ref):
  invm = 1.0 / jnp.maximum(d_ref[0, 0, :] + d_ref[1, 0, :], 1.0)
  return invm, lax.broadcast_in_dim(invm, (NP, D), (0,))


def _tc_matmul(x, w, b):
  if b is None:
    def body(x_ref, w_ref, z_out):
      z_out[...] = _dot(x_ref[...], w_ref[...])
    args = (x, w)
  else:
    def body(x_ref, w_ref, b_ref, z_out):
      z_out[...] = _dot(x_ref[...], w_ref[...]) + b_ref[...]
    args = (x, w, b)
  return pl.pallas_call(
      body, out_shape=jax.ShapeDtypeStruct((NP, D), F32))(*args)


def _tc_comb(sprev, p, deg, wn, emit_inv1):
  out_shape = [jax.ShapeDtypeStruct((NP, D), F32),
               jax.ShapeDtypeStruct((NP, D), F32)]
  if emit_inv1:
    out_shape.append(jax.ShapeDtypeStruct((NP,), F32))

  def body(s_ref, p_ref, d_ref, wn_ref, *outs):
    invm, inv = _inv_bcast(d_ref)
    if emit_inv1:
      h_out, z_out, inv1_out = outs
      inv1_out[...] = invm
    else:
      h_out, z_out = outs
    h = jnp.maximum(s_ref[...] + inv * (p_ref[0] + p_ref[1]), 0.0)
    h_out[...] = h
    z_out[...] = _dot(h, wn_ref[...])

  return pl.pallas_call(body, out_shape=out_shape)(sprev, p, deg, wn)


def _tc_final(sprev, p, deg, cpart, ws, wn, b, n):
  dout = ws.shape[1]

  def body(s_ref, p_ref, d_ref, c_ref, ws_ref, wn_ref, b_ref, out_ref):
    _, inv = _inv_bcast(d_ref)
    h = jnp.maximum(s_ref[...] + inv * (p_ref[0] + p_ref[1]), 0.0)
    cs = (c_ref[0, 0, :] + c_ref[1, 0, :]).reshape(1, NP)
    u = jnp.sum(h, axis=0, keepdims=True) * (1.0 / n)
    w = _dot(cs, h) * (1.0 / n)
    out_ref[...] = _dot(u, ws_ref[...]) + _dot(w, wn_ref[...]) + b_ref[...]

  return pl.pallas_call(
      body, out_shape=jax.ShapeDtypeStruct((1, dout), F32),
  )(sprev, p, deg, cpart, ws, wn, b)


def _pad(w):
  return jnp.pad(w, ((0, D - w.shape[0]), (0, D - w.shape[1])))


def kernel(in_feat, edge_index, W_self1, W_neigh1, b1, W_self2, W_neigh2, b2,
           W_self3, W_neigh3, b3, W_self4, W_neigh4, b4):
  n, _ = in_feat.shape
  e = edge_index.shape[1]
  nchunk = e // (NC * NS) // CHUNK
  src = edge_index[0].reshape(NC * NS, nchunk, CHUNK)
  dst = edge_index[1].reshape(NC * NS, nchunk, CHUNK)

  xp = jnp.pad(in_feat, ((0, NP - n), (0, 0)))
  ws2, wn2 = _pad(W_self2), _pad(W_neigh2)
  ws3, wn3 = _pad(W_self3), _pad(W_neigh3)
  ws4 = jnp.pad(W_self4, ((0, D - W_self4.shape[0]), (0, 0)))
  wn4 = jnp.pad(W_neigh4, ((0, D - W_neigh4.shape[0]), (0, 0)))
  b1p = b1.reshape(1, -1)
  b2p = jnp.pad(b2, (0, D - b2.shape[0])).reshape(1, -1)
  b3p = jnp.pad(b3, (0, D - b3.shape[0])).reshape(1, -1)
  b4p = b4.reshape(1, -1)

  zrows = jnp.zeros((NP // NS, D), F32)
  z1d = jnp.zeros((NP // NS,), F32)
  ones1 = jnp.ones((CHUNK,), F32)

  z1 = _tc_matmul(xp, W_neigh1, None)
  s1 = _tc_matmul(xp, W_self1, b1p)
  p1, deg = _sc_agg(e, "deg")(z1, src, dst, zrows, z1d, ones1)
  h1, z2, inv1 = _tc_comb(s1, p1, deg, wn2, True)
  s2 = _tc_matmul(h1, ws2, b2p)
  p2, cpart = _sc_agg(e, "c")(z2, inv1, src, dst, zrows, z1d)
  h2, z3 = _tc_comb(s2, p2, deg, wn3, False)
  s3 = _tc_matmul(h2, ws3, b3p)
  p3 = _sc_agg(e, "plain")(z3, src, dst, zrows, z1d)[0]
  return _tc_final(s3, p3, deg, cpart, ws4, wn4, b4p, n)
